# Initial kernel scaffold; baseline (speedup 1.0000x reference)
import jax
import jax.numpy as jnp
from jax.experimental import pallas as pl


def kernel(x, edge_index, y, W1, b1, W2, b2, W3, b3, Wl1, bl1, Wl2, bl2):
    raise NotImplementedError("write your pallas kernel here")


# speedup vs baseline: 7.1267x; 7.1267x over previous
import functools

import jax
import jax.numpy as jnp
from jax import lax
from jax.experimental import pallas as pl
from jax.experimental.pallas import tpu as pltpu
from jax.experimental.pallas import tpu_sc as plsc

N = 10000
E = 320000
FIN = 128
HD = 64
WD = 128

NC = 2
NS = 16
NW = NC * NS

NPAD = 10240
RPS = NPAD // NS
DUMMY = N + 16

CH = 128
KB = 16
EPW = 10240
NCHUNK = EPW // (KB * CH)
EPAD = NW * EPW

_f32 = jnp.float32


def _deg_body(dst_hbm, ones_hbm, z_hbm, out_hbm, idx_v, ones_v, acc, sem):
    c = lax.axis_index("c")
    s = lax.axis_index("s")
    r0 = s * RPS
    pltpu.sync_copy(z_hbm.at[pl.ds(r0, RPS)], acc.at[pl.ds(r0, RPS)])
    pltpu.sync_copy(ones_hbm, ones_v)
    plsc.subcore_barrier()
    base = (c * NS + s) * NCHUNK

    @pl.loop(0, NCHUNK)
    def _(k):
        pltpu.async_copy(dst_hbm.at[base + k], idx_v, sem).wait()

        @pl.loop(0, KB)
        def _(j):
            pltpu.sync_copy(ones_v, acc.at[idx_v.at[j]], add=True)

    plsc.subcore_barrier()
    pltpu.sync_copy(acc.at[pl.ds(r0, RPS)], out_hbm.at[c, pl.ds(r0, RPS)])


def _edge_body(h_hbm, src_hbm, dst_hbm, z_hbm, out_hbm,
               idxs_v, idxd_v, rows_v, acc, sem):
    c = lax.axis_index("c")
    s = lax.axis_index("s")
    r0 = s * RPS
    pltpu.sync_copy(z_hbm.at[pl.ds(r0, RPS)], acc.at[pl.ds(r0, RPS)])
    plsc.subcore_barrier()
    base = (c * NS + s) * NCHUNK

    @pl.loop(0, NCHUNK)
    def _(k):
        pltpu.async_copy(src_hbm.at[base + k], idxs_v, sem).wait()
        pltpu.async_copy(dst_hbm.at[base + k], idxd_v, sem).wait()

        @pl.loop(0, KB)
        def _(j):
            pltpu.async_copy(h_hbm.at[idxs_v.at[j]], rows_v, sem).wait()
            pltpu.sync_copy(rows_v, acc.at[idxd_v.at[j]], add=True)

    plsc.subcore_barrier()
    pltpu.sync_copy(acc.at[pl.ds(r0, RPS)], out_hbm.at[c, pl.ds(r0, RPS)])


def _sc_deg(dst_pad, ones_row, z_wide):
    mesh = plsc.VectorSubcoreMesh(core_axis_name="c", subcore_axis_name="s")
    k = pl.kernel(
        _deg_body,
        out_type=jax.ShapeDtypeStruct((NC, NPAD, WD), _f32),
        mesh=mesh,
        scratch_types=[
            pltpu.VMEM((KB, CH), jnp.int32),
            pltpu.VMEM((CH, WD), _f32),
            pltpu.VMEM_SHARED((NPAD, WD), _f32),
            pltpu.SemaphoreType.DMA,
        ],
    )
    return k(dst_pad, ones_row, z_wide)


def _sc_edge(h_s, src_pad, dst_pad, z_wide):
    mesh = plsc.VectorSubcoreMesh(core_axis_name="c", subcore_axis_name="s")
    k = pl.kernel(
        _edge_body,
        out_type=jax.ShapeDtypeStruct((NC, NPAD, WD), _f32),
        mesh=mesh,
        scratch_types=[
            pltpu.VMEM((KB, CH), jnp.int32),
            pltpu.VMEM((KB, CH), jnp.int32),
            pltpu.VMEM((CH, WD), _f32),
            pltpu.VMEM_SHARED((NPAD, WD), _f32),
            pltpu.SemaphoreType.DMA,
        ],
    )
    return k(h_s, src_pad, dst_pad, z_wide)


def _dot(a, b):
    return lax.dot_general(a, b, (((1,), (0,)), ((), ())),
                           precision=lax.Precision.HIGHEST,
                           preferred_element_type=_f32)


def _pre_body(x_ref, w1_ref, degp_ref, h1s_ref, dinv_ref):
    deg = degp_ref[0][:, 0:1] + degp_ref[1][:, 0:1] + 1.0
    dinv = 1.0 / jnp.sqrt(deg)
    dinv_ref[...] = jnp.broadcast_to(dinv, (NPAD, 16))
    h = _dot(x_ref[...], w1_ref[...]) * dinv
    h1s_ref[...] = jnp.concatenate([h, jnp.zeros((NPAD, WD - HD), _f32)],
                                   axis=1)


def _mid_body(p_ref, hs_ref, dinv_ref, b_ref, w_ref, out_ref):
    dv = dinv_ref[...][:, 0:1]
    agg = p_ref[0][:, :HD] + p_ref[1][:, :HD] + hs_ref[...][:, :HD]
    act = jnp.maximum(dv * agg + b_ref[0:1, :], 0.0)
    h = _dot(act, w_ref[...]) * dv
    out_ref[...] = jnp.concatenate([h, jnp.zeros((NPAD, WD - HD), _f32)],
                                   axis=1)


def _fin_body(p_ref, hs_ref, dinv_ref, b3_ref, wl1_ref, bl1_ref, wl2_ref,
              bl2_ref, y_ref, p_out_ref, loss_ref):
    dv = dinv_ref[...][:, 0:1]
    agg = p_ref[0][:, :HD] + p_ref[1][:, :HD] + hs_ref[...][:, :HD]
    act = jnp.maximum(dv * agg + b3_ref[0:1, :], 0.0)
    t = jnp.maximum(_dot(act, wl1_ref[...]) + bl1_ref[0:1, :], 0.0)
    logit = jnp.sum(t * wl2_ref[0:1, :], axis=1, keepdims=True) + bl2_ref[0:1, 0:1]
    p = jax.nn.sigmoid(logit)
    p_out_ref[...] = p
    pc = jnp.clip(p, 1e-7, 1.0 - 1e-7)
    y = y_ref[...]
    pos = jnp.sum(y) / float(N)
    w = y * (1.0 - pos) + (1.0 - y) * pos
    rowid = lax.broadcasted_iota(jnp.int32, (NPAD, 1), 0)
    mask = (rowid < N).astype(_f32)
    ll = mask * w * -(y * jnp.log(pc) + (1.0 - y) * jnp.log(1.0 - pc))
    loss_ref[...] = jnp.full((8, 128), jnp.sum(ll) / float(N), _f32)


def _tc_pre(x_pad, w1, degp):
    return pl.pallas_call(
        _pre_body,
        out_shape=[jax.ShapeDtypeStruct((NPAD, WD), _f32),
                   jax.ShapeDtypeStruct((NPAD, 16), _f32)],
    )(x_pad, w1, degp)


def _tc_mid(partials, hs, dinv, b_t, w):
    return pl.pallas_call(
        _mid_body,
        out_shape=jax.ShapeDtypeStruct((NPAD, WD), _f32),
    )(partials, hs, dinv, b_t, w)


def _tc_fin(partials, hs, dinv, b3_t, wl1, bl1_t, wl2_t, bl2_t, y_pad):
    return pl.pallas_call(
        _fin_body,
        out_shape=[jax.ShapeDtypeStruct((NPAD, 1), _f32),
                   jax.ShapeDtypeStruct((8, 128), _f32)],
    )(partials, hs, dinv, b3_t, wl1, bl1_t, wl2_t, bl2_t, y_pad)


def kernel(x, edge_index, y, W1, b1, W2, b2, W3, b3, Wl1, bl1, Wl2, bl2):
    src = edge_index[0].astype(jnp.int32)
    dst = edge_index[1].astype(jnp.int32)
    pad_e = jnp.full((EPAD - E,), DUMMY, jnp.int32)
    src_pad = jnp.concatenate([src, pad_e]).reshape(NW * NCHUNK, KB, CH)
    dst_pad = jnp.concatenate([dst, pad_e]).reshape(NW * NCHUNK, KB, CH)

    x_pad = jnp.pad(x, ((0, NPAD - N), (0, 0)))
    y_pad = jnp.pad(y, ((0, NPAD - N), (0, 0)))
    z_wide = jnp.zeros((NPAD, WD), _f32)
    ones_row = jnp.ones((CH, WD), _f32)

    b1_t = jnp.broadcast_to(b1[None, :], (8, HD))
    b2_t = jnp.broadcast_to(b2[None, :], (8, HD))
    b3_t = jnp.broadcast_to(b3[None, :], (8, HD))
    bl1_t = jnp.broadcast_to(bl1[None, :], (8, 8))
    wl2_t = jnp.broadcast_to(Wl2.reshape(1, 8), (8, 8))
    bl2_t = jnp.broadcast_to(bl2.reshape(1, 1), (8, 8))

    degp = _sc_deg(dst_pad, ones_row, z_wide)
    h1s, dinv = _tc_pre(x_pad, W1, degp)

    p1 = _sc_edge(h1s, src_pad, dst_pad, z_wide)
    h2s = _tc_mid(p1, h1s, dinv, b1_t, W2)

    p2 = _sc_edge(h2s, src_pad, dst_pad, z_wide)
    h3s = _tc_mid(p2, h2s, dinv, b2_t, W3)

    p3 = _sc_edge(h3s, src_pad, dst_pad, z_wide)
    p_pad, loss_arr = _tc_fin(p3, h3s, dinv, b3_t, Wl1, bl1_t, wl2_t, bl2_t,
                              y_pad)

    return (loss_arr[0, 0], p_pad[:N])


# speedup vs baseline: 7.2962x; 1.0238x over previous
import functools

import jax
import jax.numpy as jnp
from jax import lax
from jax.experimental import pallas as pl
from jax.experimental.pallas import tpu as pltpu
from jax.experimental.pallas import tpu_sc as plsc

N = 10000
E = 320000
FIN = 128
HD = 64
WD = 128

NC = 2
NS = 16
NW = NC * NS

NPAD = 10240
RPS = NPAD // NS
DUMMY = N + 16

CH = 128
KB = 16
EPW = 10240
NCHUNK = EPW // (KB * CH)
EPAD = NW * EPW

_f32 = jnp.float32


NBUF = 2


def _deg_body(dst_hbm, ones_hbm, z_hbm, out_hbm, idx_v, ones_v, acc,
              isem, ssem):
    c = lax.axis_index("c")
    s = lax.axis_index("s")
    r0 = s * RPS
    pltpu.sync_copy(z_hbm.at[pl.ds(r0, RPS)], acc.at[pl.ds(r0, RPS)])
    pltpu.sync_copy(ones_hbm, ones_v)
    plsc.subcore_barrier()
    base = (c * NS + s) * NCHUNK

    @pl.loop(0, NCHUNK)
    def _(k):
        pltpu.async_copy(dst_hbm.at[base + k], idx_v, isem).wait()

        @pl.loop(0, KB // NBUF)
        def _(g):
            j0 = g * NBUF
            hs = [pltpu.async_copy(ones_v, acc.at[idx_v.at[j0 + b]],
                                   ssem, add=True)
                  for b in range(NBUF)]
            for h in hs:
                h.wait()

    plsc.subcore_barrier()
    pltpu.sync_copy(acc.at[pl.ds(r0, RPS)], out_hbm.at[c, pl.ds(r0, RPS)])


def _edge_body(h_hbm, src_hbm, dst_hbm, z_hbm, out_hbm,
               idxs_v, idxd_v, rows_v, acc, gsem, ssem):
    c = lax.axis_index("c")
    s = lax.axis_index("s")
    r0 = s * RPS
    pltpu.sync_copy(z_hbm.at[pl.ds(r0, RPS)], acc.at[pl.ds(r0, RPS)])
    plsc.subcore_barrier()
    base = (c * NS + s) * NCHUNK

    @pl.loop(0, NCHUNK)
    def _(k):
        pltpu.async_copy(src_hbm.at[base + k], idxs_v, gsem).wait()
        pltpu.async_copy(dst_hbm.at[base + k], idxd_v, gsem).wait()

        @pl.loop(0, KB // NBUF)
        def _(g):
            j0 = g * NBUF
            gs = [pltpu.async_copy(h_hbm.at[idxs_v.at[j0 + b]],
                                   rows_v.at[b], gsem)
                  for b in range(NBUF)]
            for h in gs:
                h.wait()
            ss = [pltpu.async_copy(rows_v.at[b], acc.at[idxd_v.at[j0 + b]],
                                   ssem, add=True)
                  for b in range(NBUF)]
            for h in ss:
                h.wait()

    plsc.subcore_barrier()
    pltpu.sync_copy(acc.at[pl.ds(r0, RPS)], out_hbm.at[c, pl.ds(r0, RPS)])


def _sc_deg(dst_pad, ones_row, z_wide):
    mesh = plsc.VectorSubcoreMesh(core_axis_name="c", subcore_axis_name="s")
    k = pl.kernel(
        _deg_body,
        out_type=jax.ShapeDtypeStruct((NC, NPAD, WD), _f32),
        mesh=mesh,
        scratch_types=[
            pltpu.VMEM((KB, CH), jnp.int32),
            pltpu.VMEM((CH, WD), _f32),
            pltpu.VMEM_SHARED((NPAD, WD), _f32),
            pltpu.SemaphoreType.DMA,
            pltpu.SemaphoreType.DMA,
        ],
    )
    return k(dst_pad, ones_row, z_wide)


def _sc_edge(h_s, src_pad, dst_pad, z_wide):
    mesh = plsc.VectorSubcoreMesh(core_axis_name="c", subcore_axis_name="s")
    k = pl.kernel(
        _edge_body,
        out_type=jax.ShapeDtypeStruct((NC, NPAD, WD), _f32),
        mesh=mesh,
        scratch_types=[
            pltpu.VMEM((KB, CH), jnp.int32),
            pltpu.VMEM((KB, CH), jnp.int32),
            pltpu.VMEM((NBUF, CH, WD), _f32),
            pltpu.VMEM_SHARED((NPAD, WD), _f32),
            pltpu.SemaphoreType.DMA,
            pltpu.SemaphoreType.DMA,
        ],
    )
    return k(h_s, src_pad, dst_pad, z_wide)


def _dot(a, b):
    return lax.dot_general(a, b, (((1,), (0,)), ((), ())),
                           precision=lax.Precision.HIGHEST,
                           preferred_element_type=_f32)


def _pre_body(x_ref, w1_ref, degp_ref, h1s_ref, dinv_ref):
    deg = degp_ref[0][:, 0:1] + degp_ref[1][:, 0:1] + 1.0
    dinv = 1.0 / jnp.sqrt(deg)
    dinv_ref[...] = jnp.broadcast_to(dinv, (NPAD, 16))
    h = _dot(x_ref[...], w1_ref[...]) * dinv
    h1s_ref[...] = jnp.concatenate([h, jnp.zeros((NPAD, WD - HD), _f32)],
                                   axis=1)


def _mid_body(p_ref, hs_ref, dinv_ref, b_ref, w_ref, out_ref):
    dv = dinv_ref[...][:, 0:1]
    agg = p_ref[0][:, :HD] + p_ref[1][:, :HD] + hs_ref[...][:, :HD]
    act = jnp.maximum(dv * agg + b_ref[0:1, :], 0.0)
    h = _dot(act, w_ref[...]) * dv
    out_ref[...] = jnp.concatenate([h, jnp.zeros((NPAD, WD - HD), _f32)],
                                   axis=1)


def _fin_body(p_ref, hs_ref, dinv_ref, b3_ref, wl1_ref, bl1_ref, wl2_ref,
              bl2_ref, y_ref, p_out_ref, loss_ref):
    dv = dinv_ref[...][:, 0:1]
    agg = p_ref[0][:, :HD] + p_ref[1][:, :HD] + hs_ref[...][:, :HD]
    act = jnp.maximum(dv * agg + b3_ref[0:1, :], 0.0)
    t = jnp.maximum(_dot(act, wl1_ref[...]) + bl1_ref[0:1, :], 0.0)
    logit = jnp.sum(t * wl2_ref[0:1, :], axis=1, keepdims=True) + bl2_ref[0:1, 0:1]
    p = jax.nn.sigmoid(logit)
    p_out_ref[...] = p
    pc = jnp.clip(p, 1e-7, 1.0 - 1e-7)
    y = y_ref[...]
    pos = jnp.sum(y) / float(N)
    w = y * (1.0 - pos) + (1.0 - y) * pos
    rowid = lax.broadcasted_iota(jnp.int32, (NPAD, 1), 0)
    mask = (rowid < N).astype(_f32)
    ll = mask * w * -(y * jnp.log(pc) + (1.0 - y) * jnp.log(1.0 - pc))
    loss_ref[...] = jnp.full((8, 128), jnp.sum(ll) / float(N), _f32)


def _tc_pre(x_pad, w1, degp):
    return pl.pallas_call(
        _pre_body,
        out_shape=[jax.ShapeDtypeStruct((NPAD, WD), _f32),
                   jax.ShapeDtypeStruct((NPAD, 16), _f32)],
    )(x_pad, w1, degp)


def _tc_mid(partials, hs, dinv, b_t, w):
    return pl.pallas_call(
        _mid_body,
        out_shape=jax.ShapeDtypeStruct((NPAD, WD), _f32),
    )(partials, hs, dinv, b_t, w)


def _tc_fin(partials, hs, dinv, b3_t, wl1, bl1_t, wl2_t, bl2_t, y_pad):
    return pl.pallas_call(
        _fin_body,
        out_shape=[jax.ShapeDtypeStruct((NPAD, 1), _f32),
                   jax.ShapeDtypeStruct((8, 128), _f32)],
    )(partials, hs, dinv, b3_t, wl1, bl1_t, wl2_t, bl2_t, y_pad)


def kernel(x, edge_index, y, W1, b1, W2, b2, W3, b3, Wl1, bl1, Wl2, bl2):
    src = edge_index[0].astype(jnp.int32)
    dst = edge_index[1].astype(jnp.int32)
    pad_e = jnp.full((EPAD - E,), DUMMY, jnp.int32)
    src_pad = jnp.concatenate([src, pad_e]).reshape(NW * NCHUNK, KB, CH)
    dst_pad = jnp.concatenate([dst, pad_e]).reshape(NW * NCHUNK, KB, CH)

    x_pad = jnp.pad(x, ((0, NPAD - N), (0, 0)))
    y_pad = jnp.pad(y, ((0, NPAD - N), (0, 0)))
    z_wide = jnp.zeros((NPAD, WD), _f32)
    ones_row = jnp.ones((CH, WD), _f32)

    b1_t = jnp.broadcast_to(b1[None, :], (8, HD))
    b2_t = jnp.broadcast_to(b2[None, :], (8, HD))
    b3_t = jnp.broadcast_to(b3[None, :], (8, HD))
    bl1_t = jnp.broadcast_to(bl1[None, :], (8, 8))
    wl2_t = jnp.broadcast_to(Wl2.reshape(1, 8), (8, 8))
    bl2_t = jnp.broadcast_to(bl2.reshape(1, 1), (8, 8))

    degp = _sc_deg(dst_pad, ones_row, z_wide)
    h1s, dinv = _tc_pre(x_pad, W1, degp)

    p1 = _sc_edge(h1s, src_pad, dst_pad, z_wide)
    h2s = _tc_mid(p1, h1s, dinv, b1_t, W2)

    p2 = _sc_edge(h2s, src_pad, dst_pad, z_wide)
    h3s = _tc_mid(p2, h2s, dinv, b2_t, W3)

    p3 = _sc_edge(h3s, src_pad, dst_pad, z_wide)
    p_pad, loss_arr = _tc_fin(p3, h3s, dinv, b3_t, Wl1, bl1_t, wl2_t, bl2_t,
                              y_pad)

    return (loss_arr[0, 0], p_pad[:N])
